# baseline (speedup 1.0000x reference)
import functools

import jax
import jax.numpy as jnp
from jax import lax
from jax.experimental import pallas as pl
from jax.experimental.pallas import tpu as pltpu
from jax.experimental.pallas import tpu_sc as plsc

B = 128
N = 32768
LANES = 16
NUM_WORKERS = 32
ROWS_PER_W = B // NUM_WORKERS


def _row_argmax(buf):
    lane = lax.broadcasted_iota(jnp.int32, (LANES,), 0)

    def body(i, carry):
        rmax, ridx = carry
        v = buf[pl.ds(i * LANES, LANES)]
        idx = lane + i * LANES
        pred = v > rmax
        return jnp.where(pred, v, rmax), jnp.where(pred, idx, ridx)

    rmax0 = jnp.full((LANES,), -jnp.inf, jnp.float32)
    ridx0 = jnp.zeros((LANES,), jnp.int32)
    rmax, ridx = lax.fori_loop(0, N // LANES, body, (rmax0, ridx0))
    m = jnp.max(rmax)
    cand = jnp.where(rmax == m, ridx, jnp.int32(2**30))
    return jnp.min(cand)


@functools.partial(
    pl.kernel,
    out_type=jax.ShapeDtypeStruct((B, N), jnp.float32),
    mesh=plsc.VectorSubcoreMesh(core_axis_name="c", subcore_axis_name="s"),
    compiler_params=pltpu.CompilerParams(needs_layout_passes=False),
    scratch_types=[
        pltpu.VMEM((N,), jnp.float32),
        pltpu.VMEM((N,), jnp.float32),
        pltpu.VMEM((N,), jnp.float32),
        pltpu.VMEM((ROWS_PER_W, LANES), jnp.float32),
        pltpu.SemaphoreType.DMA,
        pltpu.SemaphoreType.DMA((ROWS_PER_W,)),
        pltpu.SemaphoreType.DMA,
    ],
)
def _one_hot_argmax(x_hbm, out_hbm, buf0, buf1, zbuf, fixbuf, sem_in, sem_z,
                    sem_f):
    wid = lax.axis_index("s") * 2 + lax.axis_index("c")
    row0 = wid * ROWS_PER_W

    zeros16 = jnp.zeros((LANES,), jnp.float32)

    def zfill(i, carry):
        zbuf[pl.ds(i * LANES, LANES)] = zeros16
        return carry

    lax.fori_loop(0, N // LANES, zfill, 0)

    zh = [
        pltpu.async_copy(zbuf, out_hbm.at[row0 + r], sem_z.at[r])
        for r in range(ROWS_PER_W)
    ]

    bufs = [buf0, buf1]
    handles = [None] * ROWS_PER_W
    handles[0] = pltpu.async_copy(x_hbm.at[row0], bufs[0], sem_in)
    fixups = []
    lane = lax.broadcasted_iota(jnp.int32, (LANES,), 0)
    for r in range(ROWS_PER_W):
        handles[r].wait()
        if r + 1 < ROWS_PER_W:
            handles[r + 1] = pltpu.async_copy(x_hbm.at[row0 + r + 1],
                                              bufs[(r + 1) % 2], sem_in)
        pos = _row_argmax(bufs[r % 2])
        base = (pos // LANES) * LANES
        fixbuf[r] = jnp.where(lane == pos - base, 1.0, 0.0).astype(jnp.float32)
        zh[r].wait()
        fixups.append(
            pltpu.async_copy(fixbuf.at[r],
                             out_hbm.at[row0 + r, pl.ds(base, LANES)], sem_f))
    for h in fixups:
        h.wait()


def kernel(x):
    return _one_hot_argmax(x)


# speedup vs baseline: 2.2015x; 2.2015x over previous
import functools

import jax
import jax.numpy as jnp
from jax import lax
from jax.experimental import pallas as pl
from jax.experimental.pallas import tpu as pltpu
from jax.experimental.pallas import tpu_sc as plsc

B = 128
N = 32768
LANES = 16
NUM_WORKERS = 32
ROWS_PER_W = B // NUM_WORKERS
U = 8
ITERS = N // (LANES * U)


def _row_argmax(buf):
    lane = lax.broadcasted_iota(jnp.int32, (LANES,), 0)
    neg_inf = jnp.full((LANES,), -jnp.inf, jnp.float32)
    zero_i = jnp.zeros((LANES,), jnp.int32)

    def body(i, carry):
        ivec, maxs, iters = carry
        maxs, iters = list(maxs), list(iters)
        base = i * (U * LANES)
        for u in range(U):
            v = buf[pl.ds(base + u * LANES, LANES)]
            pred = v > maxs[u]
            maxs[u] = jnp.where(pred, v, maxs[u])
            iters[u] = jnp.where(pred, ivec, iters[u])
        return ivec + 1, tuple(maxs), tuple(iters)

    _, maxs, iters = lax.fori_loop(
        0, ITERS, body, (zero_i, (neg_inf,) * U, (zero_i,) * U))

    best_m = maxs[0]
    best_p = (iters[0] * U + 0) * LANES + lane
    for u in range(1, U):
        p = (iters[u] * U + u) * LANES + lane
        better = (maxs[u] > best_m) | ((maxs[u] == best_m) & (p < best_p))
        best_m = jnp.where(better, maxs[u], best_m)
        best_p = jnp.where(better, p, best_p)
    m = jnp.max(best_m)
    cand = jnp.where(best_m == m, best_p, jnp.int32(2**30))
    return jnp.min(cand)


@functools.partial(
    pl.kernel,
    out_type=jax.ShapeDtypeStruct((B, N), jnp.float32),
    mesh=plsc.VectorSubcoreMesh(core_axis_name="c", subcore_axis_name="s"),
    compiler_params=pltpu.CompilerParams(needs_layout_passes=False),
    scratch_types=[
        pltpu.VMEM((N,), jnp.float32),
        pltpu.VMEM((N,), jnp.float32),
        pltpu.VMEM((N,), jnp.float32),
        pltpu.VMEM((ROWS_PER_W, LANES), jnp.float32),
        pltpu.SemaphoreType.DMA,
        pltpu.SemaphoreType.DMA((ROWS_PER_W,)),
        pltpu.SemaphoreType.DMA,
    ],
)
def _one_hot_argmax(x_hbm, out_hbm, buf0, buf1, zbuf, fixbuf, sem_in, sem_z,
                    sem_f):
    wid = lax.axis_index("s") * 2 + lax.axis_index("c")
    row0 = wid * ROWS_PER_W

    bufs = [buf0, buf1]
    handles = [None] * ROWS_PER_W
    handles[0] = pltpu.async_copy(x_hbm.at[row0], bufs[0], sem_in)

    zeros16 = jnp.zeros((LANES,), jnp.float32)

    def zfill(i, carry):
        base = i * (U * LANES)
        for u in range(U):
            zbuf[pl.ds(base + u * LANES, LANES)] = zeros16
        return carry

    lax.fori_loop(0, ITERS, zfill, 0)

    zh = [
        pltpu.async_copy(zbuf, out_hbm.at[row0 + r], sem_z.at[r])
        for r in range(ROWS_PER_W)
    ]

    lane = lax.broadcasted_iota(jnp.int32, (LANES,), 0)
    bases = []
    for r in range(ROWS_PER_W):
        handles[r].wait()
        if r + 1 < ROWS_PER_W:
            handles[r + 1] = pltpu.async_copy(x_hbm.at[row0 + r + 1],
                                              bufs[(r + 1) % 2], sem_in)
        pos = _row_argmax(bufs[r % 2])
        base = (pos // LANES) * LANES
        fixbuf[r] = jnp.where(lane == pos - base, 1.0, 0.0).astype(jnp.float32)
        bases.append(base)

    fixups = []
    for r in range(ROWS_PER_W):
        zh[r].wait()
        fixups.append(
            pltpu.async_copy(fixbuf.at[r],
                             out_hbm.at[row0 + r, pl.ds(bases[r], LANES)],
                             sem_f))
    for h in fixups:
        h.wait()


def kernel(x):
    return _one_hot_argmax(x)
